# Initial kernel scaffold; baseline (speedup 1.0000x reference)
import jax
import jax.numpy as jnp
from jax.experimental import pallas as pl


def kernel(x, vals, cheby_coeffs, t_mid, t_half, rows, cols):
    raise NotImplementedError("write your pallas kernel here")


# speedup vs baseline: 20.7735x; 20.7735x over previous
import jax
import jax.numpy as jnp
from jax.experimental import pallas as pl
from jax.experimental.pallas import tpu as pltpu

_N_USERS = 4096
_N_ITEMS = 4096
_B = 64
_DEGREE = 16
_UBLK = 512
_J = _N_USERS // _UBLK


def _split_stack(t):
    hi = t.astype(jnp.bfloat16)
    lo = (t - hi.astype(jnp.float32)).astype(jnp.bfloat16)
    return jnp.concatenate([hi, lo], axis=0)


def _merge(hl):
    return hl[:_B, :] + hl[_B:, :]


def _cheby_body(scal_ref, x_ref, v_ref, out_ref,
                t_prev_s, t_cur_s, t_hl_s, gv_s):
    k = pl.program_id(0)
    j = pl.program_id(1)

    @pl.when((k == 0) & (j == 0))
    def _init():
        v = v_ref[...]
        t_prev_s[...] = v
        t_cur_s[...] = v
        t_hl_s[...] = _split_stack(v)
        out_ref[...] = scal_ref[0] * v

    x_blk = x_ref[...]
    xv_hl = jax.lax.dot_general(
        t_hl_s[...], x_blk, (((1,), (1,)), ((), ())),
        preferred_element_type=jnp.float32)
    xv2 = _split_stack(_merge(xv_hl))
    g_hl = jax.lax.dot_general(
        xv2, x_blk, (((1,), (0,)), ((), ())),
        preferred_element_type=jnp.float32)
    g = _merge(g_hl)

    @pl.when(j == 0)
    def _store():
        gv_s[...] = g

    @pl.when(j > 0)
    def _accum():
        gv_s[...] += g

    @pl.when(j == _J - 1)
    def _finish():
        t_mid = scal_ref[17]
        inv_half = 1.0 / scal_ref[18]
        u = (gv_s[...] - t_mid * t_cur_s[...]) * inv_half
        alpha = jnp.where(k == 0, 1.0, 2.0)
        beta = jnp.where(k == 0, 0.0, 1.0)
        t_next = alpha * u - beta * t_prev_s[...]
        out_ref[...] += scal_ref[k + 1] * t_next
        t_prev_s[...] = t_cur_s[...]
        t_cur_s[...] = t_next
        t_hl_s[...] = _split_stack(t_next)


def _cheby_call(scal, dense_bf16, x):
    return pl.pallas_call(
        _cheby_body,
        grid=(_DEGREE, _J),
        out_shape=jax.ShapeDtypeStruct((_B, _N_ITEMS), jnp.float32),
        in_specs=[
            pl.BlockSpec(memory_space=pltpu.SMEM),
            pl.BlockSpec((_UBLK, _N_ITEMS), lambda k, j: (j, 0)),
            pl.BlockSpec((_B, _N_ITEMS), lambda k, j: (0, 0)),
        ],
        out_specs=pl.BlockSpec((_B, _N_ITEMS), lambda k, j: (0, 0)),
        scratch_shapes=[
            pltpu.VMEM((_B, _N_ITEMS), jnp.float32),
            pltpu.VMEM((_B, _N_ITEMS), jnp.float32),
            pltpu.VMEM((2 * _B, _N_ITEMS), jnp.bfloat16),
            pltpu.VMEM((_B, _N_ITEMS), jnp.float32),
        ],
        compiler_params=pltpu.CompilerParams(
            dimension_semantics=("arbitrary", "arbitrary"),
            vmem_limit_bytes=100 * 1024 * 1024,
        ),
    )(scal, dense_bf16, x)


def kernel(x, vals, cheby_coeffs, t_mid, t_half, rows, cols):
    dense = jnp.zeros((_N_USERS, _N_ITEMS), jnp.float32)
    dense = dense.at[rows, cols].add(vals)
    scal = jnp.concatenate(
        [cheby_coeffs, jnp.stack([t_mid, t_half])]).astype(jnp.float32)
    return _cheby_call(scal, dense.astype(jnp.bfloat16), x)
